# Initial kernel scaffold; baseline (speedup 1.0000x reference)
import jax
import jax.numpy as jnp
from jax.experimental import pallas as pl


def kernel(x, edge_index, params):
    raise NotImplementedError("write your pallas kernel here")


# speedup vs baseline: 1.8903x; 1.8903x over previous
import functools

import jax
import jax.numpy as jnp
from jax import lax
from jax.experimental import pallas as pl
from jax.experimental.pallas import tpu as pltpu
from jax.experimental.pallas import tpu_sc as plsc

_dot = functools.partial(jnp.dot, preferred_element_type=jnp.float32)
_dot_hi = functools.partial(
    jnp.dot, preferred_element_type=jnp.float32,
    precision=jax.lax.Precision.HIGHEST)

NCORE = 2
NSUB = 16
NW = NCORE * NSUB
C = 128
R = 1000
ZR = 128


def _pad_rows(n):
    rps = -(-n // (NSUB * ZR)) * ZR
    return rps, NSUB * rps


def _edge_layout(n, e):
    per_w = e // NW
    n_pad = _pad_rows(n)[1]
    budget = (2**21 - 1 - n_pad * C) // NSUB - 8192
    K = _pick_batch(per_w, budget // (4 * C))
    return K, per_w // K


def _pick_batch(per_worker, kmax=128):
    for k in range(min(kmax, 128), 0, -1):
        if per_worker % k == 0:
            return k
    return 1


@functools.lru_cache(maxsize=None)
def _edge_kernel(n, e, nc):
    rows, n_pad = _pad_rows(n)
    K, nb = _edge_layout(n, e)
    nz = rows // ZR
    mesh = plsc.VectorSubcoreMesh(core_axis_name="c", subcore_axis_name="s")

    def body(ab, srcr, dstr, zeros, out, src_v, dst_v, bufa, bufb, acc, sema, semb):
        cid = lax.axis_index("c")
        sid = lax.axis_index("s")
        wid = cid * NSUB + sid
        row0 = pl.multiple_of(sid * rows, 8)
        for c in range(nc):
            pltpu.sync_copy(zeros, acc.at[pl.ds(row0, rows)])
            plsc.subcore_barrier()

            def ebody(j, carry):
                pltpu.sync_copy(srcr.at[wid].at[j], src_v.at[0])
                pltpu.sync_copy(dstr.at[wid].at[j], dst_v.at[0])
                cpa = pltpu.async_copy(ab.at[c].at[dst_v.at[0]], bufa, sema)
                cpb = pltpu.async_copy(ab.at[nc + c].at[src_v.at[0]], bufb, semb)
                cpa.wait()
                cpb.wait()

                def cbody(r, c2):
                    for cc in range(C // 16):
                        s = pl.ds(cc * 16, 16)
                        v = jnp.maximum(bufa[r, s] + bufb[r, s], 0.0)
                        u = lax.bitcast_convert_type(v, jnp.uint32)
                        u = (u + jnp.uint32(0x7FFF) + ((u >> 16) & jnp.uint32(1))) \
                            & jnp.uint32(0xFFFF0000)
                        bufa[r, s] = lax.bitcast_convert_type(u, jnp.float32)
                    return c2

                lax.fori_loop(0, K, cbody, 0)
                pltpu.sync_copy(bufa, acc.at[dst_v.at[0]], add=True)
                return carry

            lax.fori_loop(0, nb, ebody, 0)
            plsc.subcore_barrier()
            sl = pl.ds(row0, rows)
            pltpu.sync_copy(acc.at[sl], out.at[cid].at[c].at[sl])
            plsc.subcore_barrier()

    return pl.kernel(
        body,
        out_type=jax.ShapeDtypeStruct((2, nc, n_pad, C), jnp.float32),
        mesh=mesh,
        scratch_types=[
            pltpu.VMEM((2, K), jnp.int32),
            pltpu.VMEM((2, K), jnp.int32),
            pltpu.VMEM((K, C), jnp.float32),
            pltpu.VMEM((K, C), jnp.float32),
            pltpu.VMEM_SHARED((n_pad, C), jnp.float32),
            pltpu.SemaphoreType.DMA,
            pltpu.SemaphoreType.DMA,
        ],
    )


@functools.lru_cache(maxsize=None)
def _count_kernel(n, e):
    CC = C
    K, nb = _edge_layout(n, e)
    rows, n_pad = _pad_rows(n)
    mesh = plsc.VectorSubcoreMesh(core_axis_name="c", subcore_axis_name="s")

    def body(dstr, zeros, out, dst_v, buf1, acc):
        cid = lax.axis_index("c")
        sid = lax.axis_index("s")
        wid = cid * NSUB + sid
        one16 = jnp.full((16,), 1.0, jnp.float32)

        def fill(r, carry):
            for cc in range(CC // 16):
                buf1[r, pl.ds(cc * 16, 16)] = one16
            return carry

        lax.fori_loop(0, K, fill, 0)
        row0 = pl.multiple_of(sid * rows, 8)
        pltpu.sync_copy(zeros, acc.at[pl.ds(row0, rows)])
        plsc.subcore_barrier()

        def ebody(j, carry):
            pltpu.sync_copy(dstr.at[wid].at[j], dst_v.at[0])
            pltpu.sync_copy(buf1, acc.at[dst_v.at[0]], add=True)
            return carry

        lax.fori_loop(0, nb, ebody, 0)
        plsc.subcore_barrier()
        sl = pl.ds(row0, rows)
        pltpu.sync_copy(acc.at[sl], out.at[cid].at[sl])

    return pl.kernel(
        body,
        out_type=jax.ShapeDtypeStruct((2, n_pad, CC), jnp.float32),
        mesh=mesh,
        scratch_types=[
            pltpu.VMEM((2, K), jnp.int32),
            pltpu.VMEM((K, CC), jnp.float32),
            pltpu.VMEM_SHARED((n_pad, CC), jnp.float32),
        ],
    )


def _colblock_matmul(h, wcat, bcat):
    n, din = h.shape
    nj = wcat.shape[0] // C
    wr = wcat.reshape(nj, C, din)
    br = bcat.reshape(nj, 1, C)

    def body(x_ref, w_ref, b_ref, o_ref):
        o_ref[0] = (
            _dot(x_ref[...], w_ref[0].T)
            + b_ref[0]
        )

    return pl.pallas_call(
        body,
        grid=(n // R, nj),
        in_specs=[
            pl.BlockSpec((R, din), lambda i, j: (i, 0)),
            pl.BlockSpec((1, C, din), lambda i, j: (j, 0, 0)),
            pl.BlockSpec((1, 1, C), lambda i, j: (j, 0, 0)),
        ],
        out_specs=pl.BlockSpec((1, R, C), lambda i, j: (j, i, 0)),
        out_shape=jax.ShapeDtypeStruct((nj, n, C), jnp.float32),
    )(h, wr, br)


def _update1(h, s3, cnt2, w2t, w1t, b1, m2):
    n, din = h.shape
    nc = s3.shape[1]
    dout = nc * C

    def body(h_ref, s_ref, c_ref, w2_ref, w1_ref, b1_ref, m2_ref, o_ref):
        cnt = c_ref[0, :, 0:1] + c_ref[1, :, 0:1]
        aggr = cnt * m2_ref[...]
        for c in range(nc):
            sc = s_ref[0, c] + s_ref[1, c]
            aggr = aggr + _dot_hi(sc, w2_ref[c * C:(c + 1) * C, :])
        if din + dout > 1000:
            u = _dot(h_ref[...], w1_ref[:din, :]) + _dot(aggr, w1_ref[din:, :])
        else:
            u = _dot(jnp.concatenate([h_ref[...], aggr], axis=1), w1_ref[...])
        o_ref[...] = jnp.maximum(u + b1_ref[...], 0.0)

    return pl.pallas_call(
        body,
        grid=(n // R,),
        in_specs=[
            pl.BlockSpec((R, din), lambda i: (i, 0)),
            pl.BlockSpec((2, nc, R, C), lambda i: (0, 0, i, 0)),
            pl.BlockSpec((2, R, C), lambda i: (0, i, 0)),
            pl.BlockSpec((dout, dout), lambda i: (0, 0)),
            pl.BlockSpec((din + dout, dout), lambda i: (0, 0)),
            pl.BlockSpec((1, dout), lambda i: (0, 0)),
            pl.BlockSpec((1, dout), lambda i: (0, 0)),
        ],
        out_specs=pl.BlockSpec((R, dout), lambda i: (i, 0)),
        out_shape=jax.ShapeDtypeStruct((n, dout), jnp.float32),
    )(h, s3, cnt2, w2t, w1t, b1, m2)


def _update2(u, w2t, b2, h=None, rwt=None, rb=None):
    n, dout = u.shape
    with_res = h is not None

    def body(*refs):
        if with_res:
            u_ref, w_ref, b_ref, h_ref, rw_ref, rb_ref, p_ref, r_ref, st_ref = refs
        else:
            u_ref, w_ref, b_ref, p_ref, st_ref = refs
        p = _dot(u_ref[...], w_ref[...]) + b_ref[...]
        p_ref[...] = p
        if with_res:
            r_ref[...] = (
                _dot(h_ref[...], rw_ref[...])
                + rb_ref[...]
            )

        @pl.when(pl.program_id(0) == 0)
        def _():
            st_ref[...] = jnp.zeros_like(st_ref)

        s1 = jnp.sum(p, axis=0, keepdims=True)
        s2 = jnp.sum(p * p, axis=0, keepdims=True)
        st_ref[...] += jnp.concatenate(
            [s1, s2, jnp.zeros((6, dout), jnp.float32)], axis=0)

    in_specs = [
        pl.BlockSpec((R, dout), lambda i: (i, 0)),
        pl.BlockSpec((dout, dout), lambda i: (0, 0)),
        pl.BlockSpec((1, dout), lambda i: (0, 0)),
    ]
    args = [u, w2t, b2]
    out_shape = [jax.ShapeDtypeStruct((n, dout), jnp.float32)]
    out_specs = [pl.BlockSpec((R, dout), lambda i: (i, 0))]
    if with_res:
        din = h.shape[1]
        in_specs += [
            pl.BlockSpec((R, din), lambda i: (i, 0)),
            pl.BlockSpec((din, dout), lambda i: (0, 0)),
            pl.BlockSpec((1, dout), lambda i: (0, 0)),
        ]
        args += [h, rwt, rb]
        out_shape.append(jax.ShapeDtypeStruct((n, dout), jnp.float32))
        out_specs.append(pl.BlockSpec((R, dout), lambda i: (i, 0)))
    out_shape.append(jax.ShapeDtypeStruct((8, dout), jnp.float32))
    out_specs.append(pl.BlockSpec((8, dout), lambda i: (0, 0)))
    return pl.pallas_call(
        body,
        grid=(n // R,),
        in_specs=in_specs,
        out_specs=out_specs,
        out_shape=out_shape,
    )(*args)


def _bn_res_relu(p, res, stats, g, b, ln=None):
    n, dout = p.shape

    def body(*refs):
        if ln is not None:
            p_ref, r_ref, st_ref, g_ref, b_ref, lg_ref, lb_ref, o_ref = refs
        else:
            p_ref, r_ref, st_ref, g_ref, b_ref, o_ref = refs
        mu = st_ref[0:1, :] / n
        var = st_ref[1:2, :] / n - mu * mu
        inv = g_ref[...] * lax.rsqrt(var + 1e-5)
        v = jnp.maximum(inv * (p_ref[...] - mu) + b_ref[...] + r_ref[...], 0.0)
        if ln is not None:
            rm = jnp.mean(v, axis=1, keepdims=True)
            rv = jnp.mean(v * v, axis=1, keepdims=True) - rm * rm
            v = lg_ref[...] * (v - rm) * lax.rsqrt(rv + 1e-5) + lb_ref[...]
        o_ref[...] = v

    in_specs = [
        pl.BlockSpec((R, dout), lambda i: (i, 0)),
        pl.BlockSpec((R, dout), lambda i: (i, 0)),
        pl.BlockSpec((8, dout), lambda i: (0, 0)),
        pl.BlockSpec((1, dout), lambda i: (0, 0)),
        pl.BlockSpec((1, dout), lambda i: (0, 0)),
    ]
    args = [p, res, stats, g, b]
    if ln is not None:
        in_specs += [
            pl.BlockSpec((1, dout), lambda i: (0, 0)),
            pl.BlockSpec((1, dout), lambda i: (0, 0)),
        ]
        args += [ln[0], ln[1]]
    return pl.pallas_call(
        body,
        grid=(n // R,),
        in_specs=in_specs,
        out_specs=pl.BlockSpec((R, dout), lambda i: (i, 0)),
        out_shape=jax.ShapeDtypeStruct((n, dout), jnp.float32),
    )(*args)


def kernel(x, edge_index, params):
    n = x.shape[0]
    e = edge_index.shape[1]
    K, nb = _edge_layout(n, e)
    src = edge_index[0].reshape(NW, nb, K)
    dst = edge_index[1].reshape(NW, nb, K)

    zrows = _pad_rows(n)[0]
    zeros = jnp.zeros((zrows, C), jnp.float32)
    cnt2 = _count_kernel(n, e)(dst, zeros)

    h = x
    nlayers = sum(1 for k in params if k.startswith("layer"))
    for i in range(1, nlayers + 1):
        p = params["layer%d" % i]
        dout, din2 = p["mW1"].shape
        din = din2 // 2
        nc = dout // C
        wcat = jnp.concatenate([p["mW1"][:, :din], p["mW1"][:, din:]], axis=0)
        bcat = jnp.concatenate([jnp.zeros((dout,), jnp.float32), p["mb1"]])
        ab = _colblock_matmul(h, wcat, bcat)
        s3 = _edge_kernel(n, e, nc)(ab, src, dst, zeros)
        w2t_r = p["mW2"].T.astype(jnp.bfloat16).astype(jnp.float32)
        u = _update1(
            h, s3, cnt2, w2t_r, p["uW1"].T, p["ub1"][None], p["mb2"][None])
        ln = (params["ln_g"][None], params["ln_b"][None]) if i == nlayers else None
        if i == 1:
            pdat, stats = _update2(u, p["uW2"].T, p["ub2"][None])
            res = h
        else:
            pdat, res, stats = _update2(
                u, p["uW2"].T, p["ub2"][None], h, p["rW"].T, p["rb"][None])
        h = _bn_res_relu(pdat, res, stats, p["bn_g"][None], p["bn_b"][None], ln=ln)
    return h
